# baseline (speedup 1.0000x reference)
import functools

import jax
import jax.numpy as jnp
import numpy as np
from jax import lax
from jax.experimental import pallas as pl
from jax.experimental.pallas import tpu as pltpu
from jax.experimental.pallas import tpu_sc as plsc

N = 10000
E = 320000
DIN = 128
H = 64

NC = 2
NS = 16
NW = NC * NS
L = 16

CK = 128
CPT = 80
EP = NW * CPT * CK
PAD_IDX = N

NP_ = 10240
RPT = NP_ // NS

NGRP = CK // L

_mesh = plsc.VectorSubcoreMesh(core_axis_name="c", subcore_axis_name="s")
_sc_params = pltpu.CompilerParams(use_tc_tiling_on_sc=False,
                                  needs_layout_passes=False)


def _wids():
    cid = lax.axis_index("c")
    sid = lax.axis_index("s")
    return cid, sid, sid * NC + cid


def _deg_body(cols_hbm, zeros_hbm, ones_hbm, out_hbm, idxc_all, onesv, acc_sh,
              sem):
    cid, sid, wid = _wids()
    base = sid * RPT
    pltpu.sync_copy(zeros_hbm.at[pl.ds(base, RPT)], acc_sh.at[pl.ds(base, RPT)])
    pltpu.sync_copy(ones_hbm, onesv)
    pltpu.sync_copy(cols_hbm.at[wid], idxc_all)
    plsc.subcore_barrier()

    @pl.loop(0, CPT)
    def _fire(c):
        pltpu.async_copy(onesv, acc_sh.at[idxc_all.at[c]], sem, add=True)

    @pl.loop(0, CPT)
    def _drain(c):
        pltpu.make_async_copy(onesv, acc_sh.at[idxc_all.at[0]], sem).wait()

    plsc.subcore_barrier()
    pltpu.sync_copy(acc_sh.at[pl.ds(base, RPT)],
                    out_hbm.at[cid, pl.ds(base, RPT)])


_deg_kernel = functools.partial(
    pl.kernel,
    out_type=jax.ShapeDtypeStruct((NC, NP_, 16), jnp.float32),
    mesh=_mesh,
    compiler_params=_sc_params,
    scratch_types=[
        pltpu.VMEM((CPT, CK), jnp.int32),
        pltpu.VMEM((CK, 16), jnp.float32),
        pltpu.VMEM_SHARED((NP_, 16), jnp.float32),
        pltpu.SemaphoreType.DMA,
    ],
)(_deg_body)


def _agg_body(g_hbm, rows_hbm, cols_hbm, zeros_hbm, out_hbm,
              idxr_all, idxc_all, gbuf, acc_sh, sem_g, sem_s):
    cid, sid, wid = _wids()
    base = sid * RPT
    pltpu.sync_copy(zeros_hbm.at[pl.ds(base, RPT)], acc_sh.at[pl.ds(base, RPT)])
    pltpu.sync_copy(rows_hbm.at[wid], idxr_all)
    pltpu.sync_copy(cols_hbm.at[wid], idxc_all)
    plsc.subcore_barrier()

    def _gather_start(c, b):
        pltpu.async_copy(g_hbm.at[idxr_all.at[c]], gbuf.at[b], sem_g.at[b])

    def _gather_wait(b):
        pltpu.make_async_copy(g_hbm.at[idxr_all.at[0]], gbuf.at[b],
                              sem_g.at[b]).wait()

    def _scatter_start(c, b):
        pltpu.async_copy(gbuf.at[b], acc_sh.at[idxc_all.at[c]], sem_s.at[b],
                         add=True)

    def _scatter_wait(b):
        pltpu.make_async_copy(gbuf.at[b], acc_sh.at[idxc_all.at[0]],
                              sem_s.at[b]).wait()

    _gather_start(0, 0)
    _gather_start(1, 1)

    @pl.loop(0, CPT // 4)
    def _round(r):
        for b in range(4):
            c = r * 4 + b
            b2 = (b + 2) % 4
            @pl.when(c >= 2)
            def _():
                _scatter_wait(b2)

            @pl.when(c + 2 < CPT)
            def _():
                _gather_start(c + 2, b2)

            _gather_wait(b)
            _scatter_start(c, b)

    _scatter_wait((CPT - 2) % 4)
    _scatter_wait((CPT - 1) % 4)
    plsc.subcore_barrier()
    pltpu.sync_copy(acc_sh.at[pl.ds(base, RPT)],
                    out_hbm.at[cid, pl.ds(base, RPT)])


_agg_kernel = functools.partial(
    pl.kernel,
    out_type=jax.ShapeDtypeStruct((NC, NP_, H), jnp.float32),
    mesh=_mesh,
    compiler_params=_sc_params,
    scratch_types=[
        pltpu.VMEM((CPT, CK), jnp.int32),
        pltpu.VMEM((CPT, CK), jnp.int32),
        pltpu.VMEM((4, CK, H), jnp.float32),
        pltpu.VMEM_SHARED((NP_, H), jnp.float32),
        pltpu.SemaphoreType.DMA((4,)),
        pltpu.SemaphoreType.DMA((4,)),
    ],
)(_agg_body)


def _edge_body(a_hbm, b_hbm, rows_hbm, cols_hbm, wq_hbm, bc2_hbm, out_hbm,
               idxr_all, idxc_all, ebufa, wv, bv, outv_all,
               sem_a, sem_b):
    cid, sid, wid = _wids()
    pltpu.sync_copy(wq_hbm, wv)
    pltpu.sync_copy(bc2_hbm, bv)
    pltpu.sync_copy(rows_hbm.at[wid], idxr_all)
    pltpu.sync_copy(cols_hbm.at[wid], idxc_all)

    def _start_a(c, b):
        pltpu.async_copy(a_hbm.at[idxr_all.at[c]], ebufa.at[b], sem_a.at[b])

    def _wait_a(b):
        pltpu.make_async_copy(a_hbm.at[idxr_all.at[0]], ebufa.at[b],
                              sem_a.at[b]).wait()

    def _start_b(c, b):
        pltpu.async_copy(b_hbm.at[idxc_all.at[c]], ebufa.at[b], sem_b.at[b],
                         add=True)

    def _wait_b(b):
        pltpu.make_async_copy(b_hbm.at[idxc_all.at[0]], ebufa.at[b],
                              sem_b.at[b]).wait()

    _start_a(0, 0)
    _start_a(1, 1)
    _wait_a(0)
    _start_b(0, 0)

    wq = [wv.at[k][...] for k in range(H // L)]
    last_lane = lax.iota(jnp.int32, L) == (L - 1)

    @pl.loop(0, CPT // 2)
    def _round(r):
        for b in range(2):
            c = r * 2 + b
            _wait_b(b)
            cbase = c * CK

            @pl.loop(0, CK, unroll=4)
            def _edge(e):
                acc = None
                for k in range(H // L):
                    t = ebufa.at[b][e, pl.ds(k * L, L)]
                    t = jnp.maximum(t, 0.0) * wq[k]
                    acc = t if acc is None else acc + t
                cums = plsc.cumsum(acc)
                pos = jnp.full((L,), 0, jnp.int32) + (cbase + e)
                plsc.store_scatter(outv_all, [pos], cums, mask=last_lane)

            @pl.when(c + 2 < CPT)
            def _():
                _start_a(c + 2, b)

            @pl.when(c + 1 < CPT)
            def _():
                _wait_a(1 - b)
                _start_b(c + 1, 1 - b)

    bias = bv[...]

    @pl.loop(0, CPT * CK // L)
    def _sig(i):
        s = outv_all[pl.ds(i * L, L)] + bias
        outv_all[pl.ds(i * L, L)] = 1.0 / (1.0 + jnp.exp(-s))

    pltpu.sync_copy(outv_all, out_hbm.at[wid])


_edge_kernel = functools.partial(
    pl.kernel,
    out_type=jax.ShapeDtypeStruct((NW, CPT * CK), jnp.float32),
    mesh=_mesh,
    compiler_params=_sc_params,
    scratch_types=[
        pltpu.VMEM((CPT, CK), jnp.int32),
        pltpu.VMEM((CPT, CK), jnp.int32),
        pltpu.VMEM((2, CK, H), jnp.float32),
        pltpu.VMEM((H // L, L), jnp.float32),
        pltpu.VMEM((L,), jnp.float32),
        pltpu.VMEM((CPT * CK,), jnp.float32),
        pltpu.SemaphoreType.DMA((2,)),
        pltpu.SemaphoreType.DMA((2,)),
    ],
)(_edge_body)


BN = 512


def _dis_block(dega, degb):
    deg = dega[:, :1] + degb[:, :1] + 1.0
    return lax.rsqrt(deg)


def _tc1_body(x_ref, w1_ref, dega_ref, degb_ref, g1_ref):
    dis = _dis_block(dega_ref[...], degb_ref[...])
    g1_ref[...] = jnp.dot(x_ref[...], w1_ref[...],
                          preferred_element_type=jnp.float32) * dis


def _tc2_body(g1_ref, s1a_ref, s1b_ref, dega_ref, degb_ref, b1_ref, w2_ref,
              g2_ref):
    dis = _dis_block(dega_ref[...], degb_ref[...])
    h1 = dis * (s1a_ref[...] + s1b_ref[...] + g1_ref[...]) + b1_ref[...]
    h1 = jnp.maximum(h1, 0.0)
    g2_ref[...] = jnp.dot(h1, w2_ref[...],
                          preferred_element_type=jnp.float32) * dis


def _tc3_body(g2_ref, s2a_ref, s2b_ref, dega_ref, degb_ref, b2_ref, wc1_ref,
              bc1_ref, a_ref, b_ref):
    dis = _dis_block(dega_ref[...], degb_ref[...])
    h2 = dis * (s2a_ref[...] + s2b_ref[...] + g2_ref[...]) + b2_ref[...]
    wc1 = wc1_ref[...]
    a_ref[...] = jnp.dot(h2, wc1[:H], preferred_element_type=jnp.float32) \
        + bc1_ref[...]
    b_ref[...] = jnp.dot(h2, wc1[H:], preferred_element_type=jnp.float32)


def _row_spec(width):
    return pl.BlockSpec((BN, width), lambda i: (i, 0))


def _full_spec(rows, cols):
    return pl.BlockSpec((rows, cols), lambda i: (0, 0))


_GRID = (NP_ // BN,)

_tc1 = pl.pallas_call(
    _tc1_body,
    grid=_GRID,
    in_specs=[_row_spec(DIN), _full_spec(DIN, H), _row_spec(16), _row_spec(16)],
    out_specs=_row_spec(H),
    out_shape=jax.ShapeDtypeStruct((NP_, H), jnp.float32),
)

_tc2 = pl.pallas_call(
    _tc2_body,
    grid=_GRID,
    in_specs=[_row_spec(H), _row_spec(H), _row_spec(H), _row_spec(16),
              _row_spec(16), _full_spec(1, H), _full_spec(H, H)],
    out_specs=_row_spec(H),
    out_shape=jax.ShapeDtypeStruct((NP_, H), jnp.float32),
)

_tc3 = pl.pallas_call(
    _tc3_body,
    grid=_GRID,
    in_specs=[_row_spec(H), _row_spec(H), _row_spec(H), _row_spec(16),
              _row_spec(16), _full_spec(1, H), _full_spec(2 * H, H),
              _full_spec(1, H)],
    out_specs=[_row_spec(H), _row_spec(H)],
    out_shape=[jax.ShapeDtypeStruct((NP_, H), jnp.float32),
               jax.ShapeDtypeStruct((NP_, H), jnp.float32)],
)


def kernel(x, edge_index, W1, b1, W2, b2, Wc1, bc1, Wc2, bc2):
    row = edge_index[0]
    col = edge_index[1]
    pad = EP - E
    pad_idx = N + (jnp.arange(pad, dtype=jnp.int32) % (NP_ - N))
    rows3 = jnp.concatenate([row, pad_idx]).reshape(NW, CPT, CK)
    cols3 = jnp.concatenate([col, pad_idx]).reshape(NW, CPT, CK)

    x_p = jnp.zeros((NP_, DIN), jnp.float32).at[:N].set(x)
    ones16 = jnp.ones((CK, 16), jnp.float32)
    zer16 = jnp.zeros((NP_, 16), jnp.float32)
    zer64 = jnp.zeros((NP_, H), jnp.float32)
    wq = Wc2.reshape(H // L, L)
    bc2b = jnp.broadcast_to(bc2.reshape(1), (L,))

    deg = _deg_kernel(cols3, zer16, ones16)
    dega, degb = deg[0], deg[1]

    g1 = _tc1(x_p, W1, dega, degb)
    s1 = _agg_kernel(g1, rows3, cols3, zer64)
    g2 = _tc2(g1, s1[0], s1[1], dega, degb, b1.reshape(1, H), W2)
    s2 = _agg_kernel(g2, rows3, cols3, zer64)
    a_nodes, b_nodes = _tc3(g2, s2[0], s2[1], dega, degb, b2.reshape(1, H),
                            Wc1, bc1.reshape(1, H))

    out3 = _edge_kernel(a_nodes, b_nodes, rows3, cols3, wq, bc2b)
    return out3.reshape(EP)[:E]


# speedup vs baseline: 1.1252x; 1.1252x over previous
import functools

import jax
import jax.numpy as jnp
import numpy as np
from jax import lax
from jax.experimental import pallas as pl
from jax.experimental.pallas import tpu as pltpu
from jax.experimental.pallas import tpu_sc as plsc

N = 10000
E = 320000
DIN = 128
H = 64

NC = 2
NS = 16
NW = NC * NS
L = 16

CK = 128
CPT = 80
EP = NW * CPT * CK
PAD_IDX = N

NP_ = 10240
RPT = NP_ // NS

NGRP = CK // L

_mesh = plsc.VectorSubcoreMesh(core_axis_name="c", subcore_axis_name="s")
_sc_params = pltpu.CompilerParams(use_tc_tiling_on_sc=False,
                                  needs_layout_passes=False)


def _wids():
    cid = lax.axis_index("c")
    sid = lax.axis_index("s")
    return cid, sid, sid * NC + cid


def _deg_body(cols_hbm, zeros_hbm, ones_hbm, out_hbm, idxc_all, onesv, acc_sh,
              sem):
    cid, sid, wid = _wids()
    base = sid * RPT
    pltpu.sync_copy(zeros_hbm.at[pl.ds(base, RPT)], acc_sh.at[pl.ds(base, RPT)])
    pltpu.sync_copy(ones_hbm, onesv)
    pltpu.sync_copy(cols_hbm.at[wid], idxc_all)
    plsc.subcore_barrier()

    @pl.loop(0, CPT)
    def _fire(c):
        pltpu.async_copy(onesv, acc_sh.at[idxc_all.at[c]], sem, add=True)

    @pl.loop(0, CPT)
    def _drain(c):
        pltpu.make_async_copy(onesv, acc_sh.at[idxc_all.at[0]], sem).wait()

    plsc.subcore_barrier()
    pltpu.sync_copy(acc_sh.at[pl.ds(base, RPT)],
                    out_hbm.at[cid, pl.ds(base, RPT)])


_deg_kernel = functools.partial(
    pl.kernel,
    out_type=jax.ShapeDtypeStruct((NC, NP_, 16), jnp.float32),
    mesh=_mesh,
    compiler_params=_sc_params,
    scratch_types=[
        pltpu.VMEM((CPT, CK), jnp.int32),
        pltpu.VMEM((CK, 16), jnp.float32),
        pltpu.VMEM_SHARED((NP_, 16), jnp.float32),
        pltpu.SemaphoreType.DMA,
    ],
)(_deg_body)


def _agg_body(g_hbm, rows_hbm, cols_hbm, zeros_hbm, out_hbm,
              idxr_all, idxc_all, gbuf, acc_sh, sem_g, sem_s):
    cid, sid, wid = _wids()
    base = sid * RPT
    pltpu.sync_copy(zeros_hbm.at[pl.ds(base, RPT)], acc_sh.at[pl.ds(base, RPT)])
    pltpu.sync_copy(rows_hbm.at[wid], idxr_all)
    pltpu.sync_copy(cols_hbm.at[wid], idxc_all)
    plsc.subcore_barrier()

    def _gather_start(c, b):
        pltpu.async_copy(g_hbm.at[idxr_all.at[c]], gbuf.at[b], sem_g.at[b])

    def _gather_wait(b):
        pltpu.make_async_copy(g_hbm.at[idxr_all.at[0]], gbuf.at[b],
                              sem_g.at[b]).wait()

    def _scatter_start(c, b):
        pltpu.async_copy(gbuf.at[b], acc_sh.at[idxc_all.at[c]], sem_s.at[b],
                         add=True)

    def _scatter_wait(b):
        pltpu.make_async_copy(gbuf.at[b], acc_sh.at[idxc_all.at[0]],
                              sem_s.at[b]).wait()

    _gather_start(0, 0)
    _gather_start(1, 1)

    @pl.loop(0, CPT // 4)
    def _round(r):
        for b in range(4):
            c = r * 4 + b
            b2 = (b + 2) % 4
            @pl.when(c >= 2)
            def _():
                _scatter_wait(b2)

            @pl.when(c + 2 < CPT)
            def _():
                _gather_start(c + 2, b2)

            _gather_wait(b)
            _scatter_start(c, b)

    _scatter_wait((CPT - 2) % 4)
    _scatter_wait((CPT - 1) % 4)
    plsc.subcore_barrier()
    pltpu.sync_copy(acc_sh.at[pl.ds(base, RPT)],
                    out_hbm.at[cid, pl.ds(base, RPT)])


_agg_kernel = functools.partial(
    pl.kernel,
    out_type=jax.ShapeDtypeStruct((NC, NP_, H), jnp.float32),
    mesh=_mesh,
    compiler_params=_sc_params,
    scratch_types=[
        pltpu.VMEM((CPT, CK), jnp.int32),
        pltpu.VMEM((CPT, CK), jnp.int32),
        pltpu.VMEM((4, CK, H), jnp.float32),
        pltpu.VMEM_SHARED((NP_, H), jnp.float32),
        pltpu.SemaphoreType.DMA((4,)),
        pltpu.SemaphoreType.DMA((4,)),
    ],
)(_agg_body)


def _edge_body(a_hbm, b_hbm, rows_hbm, cols_hbm, wq_hbm, bc2_hbm, out_hbm,
               idxr_all, idxc_all, ebufa, ebufb, wv, bv, outv_all,
               sem_a, sem_b):
    cid, sid, wid = _wids()
    pltpu.sync_copy(wq_hbm, wv)
    pltpu.sync_copy(bc2_hbm, bv)
    pltpu.sync_copy(rows_hbm.at[wid], idxr_all)
    pltpu.sync_copy(cols_hbm.at[wid], idxc_all)

    NB = 4

    def _start(c, b):
        pltpu.async_copy(a_hbm.at[idxr_all.at[c]], ebufa.at[b], sem_a.at[b])
        pltpu.async_copy(b_hbm.at[idxc_all.at[c]], ebufb.at[b], sem_b.at[b])

    def _wait(b):
        pltpu.make_async_copy(a_hbm.at[idxr_all.at[0]], ebufa.at[b],
                              sem_a.at[b]).wait()
        pltpu.make_async_copy(b_hbm.at[idxc_all.at[0]], ebufb.at[b],
                              sem_b.at[b]).wait()

    for b in range(NB):
        _start(b, b)

    wq = [wv.at[k][...] for k in range(H // L)]
    last_lane = lax.iota(jnp.int32, L) == (L - 1)

    @pl.loop(0, CPT // NB)
    def _round(r):
        for b in range(NB):
            c = r * NB + b
            _wait(b)
            cbase = c * CK

            @pl.loop(0, CK, unroll=4)
            def _edge(e):
                acc = None
                for k in range(H // L):
                    va = ebufa.at[b][e, pl.ds(k * L, L)]
                    vb = ebufb.at[b][e, pl.ds(k * L, L)]
                    t = jnp.maximum(va + vb, 0.0) * wq[k]
                    acc = t if acc is None else acc + t
                cums = plsc.cumsum(acc)
                pos = jnp.full((L,), 0, jnp.int32) + (cbase + e)
                plsc.store_scatter(outv_all, [pos], cums, mask=last_lane)

            @pl.when(c + NB < CPT)
            def _():
                _start(c + NB, b)

    bias = bv[...]

    @pl.loop(0, CPT * CK // L)
    def _sig(i):
        s = outv_all[pl.ds(i * L, L)] + bias
        outv_all[pl.ds(i * L, L)] = 1.0 / (1.0 + jnp.exp(-s))

    pltpu.sync_copy(outv_all, out_hbm.at[wid])


_edge_kernel = functools.partial(
    pl.kernel,
    out_type=jax.ShapeDtypeStruct((NW, CPT * CK), jnp.float32),
    mesh=_mesh,
    compiler_params=_sc_params,
    scratch_types=[
        pltpu.VMEM((CPT, CK), jnp.int32),
        pltpu.VMEM((CPT, CK), jnp.int32),
        pltpu.VMEM((4, CK, H), jnp.float32),
        pltpu.VMEM((4, CK, H), jnp.float32),
        pltpu.VMEM((H // L, L), jnp.float32),
        pltpu.VMEM((L,), jnp.float32),
        pltpu.VMEM((CPT * CK,), jnp.float32),
        pltpu.SemaphoreType.DMA((4,)),
        pltpu.SemaphoreType.DMA((4,)),
    ],
)(_edge_body)


BN = 512


def _dis_block(dega, degb):
    deg = dega[:, :1] + degb[:, :1] + 1.0
    return lax.rsqrt(deg)


def _tc1a_body(x_ref, w1_ref, xw_ref):
    xw_ref[...] = jnp.dot(x_ref[...], w1_ref[...],
                          preferred_element_type=jnp.float32)


def _tc1b_body(xw_ref, dega_ref, degb_ref, g1_ref):
    dis = _dis_block(dega_ref[...], degb_ref[...])
    g1_ref[...] = xw_ref[...] * dis


def _tc2_body(g1_ref, s1a_ref, s1b_ref, dega_ref, degb_ref, b1_ref, w2_ref,
              g2_ref):
    dis = _dis_block(dega_ref[...], degb_ref[...])
    h1 = dis * (s1a_ref[...] + s1b_ref[...] + g1_ref[...]) + b1_ref[...]
    h1 = jnp.maximum(h1, 0.0)
    g2_ref[...] = jnp.dot(h1, w2_ref[...],
                          preferred_element_type=jnp.float32) * dis


def _tc3_body(g2_ref, s2a_ref, s2b_ref, dega_ref, degb_ref, b2_ref, wc1_ref,
              bc1_ref, a_ref, b_ref):
    dis = _dis_block(dega_ref[...], degb_ref[...])
    h2 = dis * (s2a_ref[...] + s2b_ref[...] + g2_ref[...]) + b2_ref[...]
    wc1 = wc1_ref[...]
    a_ref[...] = jnp.dot(h2, wc1[:H], preferred_element_type=jnp.float32) \
        + bc1_ref[...]
    b_ref[...] = jnp.dot(h2, wc1[H:], preferred_element_type=jnp.float32)


def _row_spec(width):
    return pl.BlockSpec((BN, width), lambda i: (i, 0))


def _full_spec(rows, cols):
    return pl.BlockSpec((rows, cols), lambda i: (0, 0))


_GRID = (NP_ // BN,)

_tc1a = pl.pallas_call(
    _tc1a_body,
    grid=_GRID,
    in_specs=[_row_spec(DIN), _full_spec(DIN, H)],
    out_specs=_row_spec(H),
    out_shape=jax.ShapeDtypeStruct((NP_, H), jnp.float32),
)

_tc1b = pl.pallas_call(
    _tc1b_body,
    grid=_GRID,
    in_specs=[_row_spec(H), _row_spec(16), _row_spec(16)],
    out_specs=_row_spec(H),
    out_shape=jax.ShapeDtypeStruct((NP_, H), jnp.float32),
)

_tc2 = pl.pallas_call(
    _tc2_body,
    grid=_GRID,
    in_specs=[_row_spec(H), _row_spec(H), _row_spec(H), _row_spec(16),
              _row_spec(16), _full_spec(1, H), _full_spec(H, H)],
    out_specs=_row_spec(H),
    out_shape=jax.ShapeDtypeStruct((NP_, H), jnp.float32),
)

_tc3 = pl.pallas_call(
    _tc3_body,
    grid=_GRID,
    in_specs=[_row_spec(H), _row_spec(H), _row_spec(H), _row_spec(16),
              _row_spec(16), _full_spec(1, H), _full_spec(2 * H, H),
              _full_spec(1, H)],
    out_specs=[_row_spec(H), _row_spec(H)],
    out_shape=[jax.ShapeDtypeStruct((NP_, H), jnp.float32),
               jax.ShapeDtypeStruct((NP_, H), jnp.float32)],
)


def kernel(x, edge_index, W1, b1, W2, b2, Wc1, bc1, Wc2, bc2):
    row = edge_index[0]
    col = edge_index[1]
    pad = EP - E
    pad_idx = N + (jnp.arange(pad, dtype=jnp.int32) % (NP_ - N))
    rows3 = jnp.concatenate([row, pad_idx]).reshape(NW, CPT, CK)
    cols3 = jnp.concatenate([col, pad_idx]).reshape(NW, CPT, CK)

    x_p = jnp.zeros((NP_, DIN), jnp.float32).at[:N].set(x)
    ones16 = jnp.ones((CK, 16), jnp.float32)
    zer16 = jnp.zeros((NP_, 16), jnp.float32)
    zer64 = jnp.zeros((NP_, H), jnp.float32)
    wq = Wc2.reshape(H // L, L)
    bc2b = jnp.broadcast_to(bc2.reshape(1), (L,))

    xw = _tc1a(x_p, W1)
    deg = _deg_kernel(cols3, zer16, ones16)
    dega, degb = deg[0], deg[1]

    g1 = _tc1b(xw, dega, degb)
    s1 = _agg_kernel(g1, rows3, cols3, zer64)
    g2 = _tc2(g1, s1[0], s1[1], dega, degb, b1.reshape(1, H), W2)
    s2 = _agg_kernel(g2, rows3, cols3, zer64)
    a_nodes, b_nodes = _tc3(g2, s2[0], s2[1], dega, degb, b2.reshape(1, H),
                            Wc1, bc1.reshape(1, H))

    out3 = _edge_kernel(a_nodes, b_nodes, rows3, cols3, wq, bc2b)
    return out3.reshape(EP)[:E]


# speedup vs baseline: 1.2482x; 1.1093x over previous
import functools

import jax
import jax.numpy as jnp
import numpy as np
from jax import lax
from jax.experimental import pallas as pl
from jax.experimental.pallas import tpu as pltpu
from jax.experimental.pallas import tpu_sc as plsc

N = 10000
E = 320000
DIN = 128
H = 64

NC = 2
NS = 16
NW = NC * NS
L = 16

CK = 128
CPT = 80
EP = NW * CPT * CK
PAD_IDX = N

NP_ = 10240
RPT = NP_ // NS

NGRP = CK // L

_mesh = plsc.VectorSubcoreMesh(core_axis_name="c", subcore_axis_name="s")
_sc_params = pltpu.CompilerParams(use_tc_tiling_on_sc=False,
                                  needs_layout_passes=False)


def _wids():
    cid = lax.axis_index("c")
    sid = lax.axis_index("s")
    return cid, sid, sid * NC + cid


def _deg_body(cols_hbm, zeros_hbm, ones_hbm, outa_hbm, outb_hbm, idxc_all,
              onesv, acc_sh, sem):
    cid, sid, wid = _wids()
    base = sid * RPT
    pltpu.sync_copy(zeros_hbm.at[pl.ds(base, RPT)], acc_sh.at[pl.ds(base, RPT)])
    pltpu.sync_copy(ones_hbm, onesv)
    pltpu.sync_copy(cols_hbm.at[wid], idxc_all)
    plsc.subcore_barrier()

    @pl.loop(0, CPT)
    def _fire(c):
        pltpu.async_copy(onesv, acc_sh.at[idxc_all.at[c]], sem, add=True)

    @pl.loop(0, CPT)
    def _drain(c):
        pltpu.make_async_copy(onesv, acc_sh.at[idxc_all.at[0]], sem).wait()

    plsc.subcore_barrier()

    @pl.when(cid == 0)
    def _():
        pltpu.sync_copy(acc_sh.at[pl.ds(base, RPT)],
                        outa_hbm.at[pl.ds(base, RPT)])

    @pl.when(cid == 1)
    def _():
        pltpu.sync_copy(acc_sh.at[pl.ds(base, RPT)],
                        outb_hbm.at[pl.ds(base, RPT)])


_deg_kernel = functools.partial(
    pl.kernel,
    out_type=[jax.ShapeDtypeStruct((NP_, 16), jnp.float32),
              jax.ShapeDtypeStruct((NP_, 16), jnp.float32)],
    mesh=_mesh,
    compiler_params=_sc_params,
    scratch_types=[
        pltpu.VMEM((CPT, CK), jnp.int32),
        pltpu.VMEM((CK, 16), jnp.float32),
        pltpu.VMEM_SHARED((NP_, 16), jnp.float32),
        pltpu.SemaphoreType.DMA,
    ],
)(_deg_body)


def _agg_body(g_hbm, rows_hbm, cols_hbm, zeros_hbm, outa_hbm, outb_hbm,
              idxr_all, idxc_all, gbuf, acc_sh, sem_g, sem_s):
    cid, sid, wid = _wids()
    base = sid * RPT
    pltpu.sync_copy(zeros_hbm.at[pl.ds(base, RPT)], acc_sh.at[pl.ds(base, RPT)])
    pltpu.sync_copy(rows_hbm.at[wid], idxr_all)
    pltpu.sync_copy(cols_hbm.at[wid], idxc_all)
    plsc.subcore_barrier()

    def _gather_start(c, b):
        pltpu.async_copy(g_hbm.at[idxr_all.at[c]], gbuf.at[b], sem_g.at[b])

    def _gather_wait(b):
        pltpu.make_async_copy(g_hbm.at[idxr_all.at[0]], gbuf.at[b],
                              sem_g.at[b]).wait()

    def _scatter_start(c, b):
        pltpu.async_copy(gbuf.at[b], acc_sh.at[idxc_all.at[c]], sem_s.at[b],
                         add=True)

    def _scatter_wait(b):
        pltpu.make_async_copy(gbuf.at[b], acc_sh.at[idxc_all.at[0]],
                              sem_s.at[b]).wait()

    _gather_start(0, 0)
    _gather_start(1, 1)

    @pl.loop(0, CPT // 4)
    def _round(r):
        for b in range(4):
            c = r * 4 + b
            b2 = (b + 2) % 4
            @pl.when(c >= 2)
            def _():
                _scatter_wait(b2)

            @pl.when(c + 2 < CPT)
            def _():
                _gather_start(c + 2, b2)

            _gather_wait(b)
            _scatter_start(c, b)

    _scatter_wait((CPT - 2) % 4)
    _scatter_wait((CPT - 1) % 4)
    plsc.subcore_barrier()

    @pl.when(cid == 0)
    def _():
        pltpu.sync_copy(acc_sh.at[pl.ds(base, RPT)],
                        outa_hbm.at[pl.ds(base, RPT)])

    @pl.when(cid == 1)
    def _():
        pltpu.sync_copy(acc_sh.at[pl.ds(base, RPT)],
                        outb_hbm.at[pl.ds(base, RPT)])


_agg_kernel = functools.partial(
    pl.kernel,
    out_type=[jax.ShapeDtypeStruct((NP_, H), jnp.float32),
              jax.ShapeDtypeStruct((NP_, H), jnp.float32)],
    mesh=_mesh,
    compiler_params=_sc_params,
    scratch_types=[
        pltpu.VMEM((CPT, CK), jnp.int32),
        pltpu.VMEM((CPT, CK), jnp.int32),
        pltpu.VMEM((4, CK, H), jnp.float32),
        pltpu.VMEM_SHARED((NP_, H), jnp.float32),
        pltpu.SemaphoreType.DMA((4,)),
        pltpu.SemaphoreType.DMA((4,)),
    ],
)(_agg_body)


def _edge_body(a_hbm, b_hbm, rows_hbm, cols_hbm, wq_hbm, bc2_hbm, out_hbm,
               idxr_all, idxc_all, ebufa, ebufb, wv, bv, outv_all,
               sem_a, sem_b):
    cid, sid, wid = _wids()
    pltpu.sync_copy(wq_hbm, wv)
    pltpu.sync_copy(bc2_hbm, bv)
    pltpu.sync_copy(rows_hbm.at[wid], idxr_all)
    pltpu.sync_copy(cols_hbm.at[wid], idxc_all)

    NB = 4

    def _start(c, b):
        pltpu.async_copy(a_hbm.at[idxr_all.at[c]], ebufa.at[b], sem_a.at[b])
        pltpu.async_copy(b_hbm.at[idxc_all.at[c]], ebufb.at[b], sem_b.at[b])

    def _wait(b):
        pltpu.make_async_copy(a_hbm.at[idxr_all.at[0]], ebufa.at[b],
                              sem_a.at[b]).wait()
        pltpu.make_async_copy(b_hbm.at[idxc_all.at[0]], ebufb.at[b],
                              sem_b.at[b]).wait()

    for b in range(NB):
        _start(b, b)

    wq = [wv.at[k][...] for k in range(H // L)]
    last_lane = lax.iota(jnp.int32, L) == (L - 1)

    @pl.loop(0, CPT // NB)
    def _round(r):
        for b in range(NB):
            c = r * NB + b
            _wait(b)
            cbase = c * CK

            @pl.loop(0, CK, unroll=4)
            def _edge(e):
                acc = None
                for k in range(H // L):
                    va = ebufa.at[b][e, pl.ds(k * L, L)]
                    vb = ebufb.at[b][e, pl.ds(k * L, L)]
                    t = jnp.maximum(va + vb, 0.0) * wq[k]
                    acc = t if acc is None else acc + t
                cums = plsc.cumsum(acc)
                pos = jnp.full((L,), 0, jnp.int32) + (cbase + e)
                plsc.store_scatter(outv_all, [pos], cums, mask=last_lane)

            @pl.when(c + NB < CPT)
            def _():
                _start(c + NB, b)

    bias = bv[...]

    @pl.loop(0, CPT * CK // L)
    def _sig(i):
        s = outv_all[pl.ds(i * L, L)] + bias
        outv_all[pl.ds(i * L, L)] = 1.0 / (1.0 + jnp.exp(-s))

    pltpu.sync_copy(outv_all, out_hbm.at[wid])


_edge_kernel = functools.partial(
    pl.kernel,
    out_type=jax.ShapeDtypeStruct((NW, CPT * CK), jnp.float32),
    mesh=_mesh,
    compiler_params=_sc_params,
    scratch_types=[
        pltpu.VMEM((CPT, CK), jnp.int32),
        pltpu.VMEM((CPT, CK), jnp.int32),
        pltpu.VMEM((4, CK, H), jnp.float32),
        pltpu.VMEM((4, CK, H), jnp.float32),
        pltpu.VMEM((H // L, L), jnp.float32),
        pltpu.VMEM((L,), jnp.float32),
        pltpu.VMEM((CPT * CK,), jnp.float32),
        pltpu.SemaphoreType.DMA((4,)),
        pltpu.SemaphoreType.DMA((4,)),
    ],
)(_edge_body)


BN = 2048


def _dis_block(dega, degb):
    deg = dega[:, :1] + degb[:, :1] + 1.0
    return lax.rsqrt(deg)


def _tc1a_body(x_ref, w1_ref, xw_ref):
    xw_ref[...] = jnp.dot(x_ref[...], w1_ref[...],
                          preferred_element_type=jnp.float32)


def _tc1b_body(xw_ref, dega_ref, degb_ref, g1_ref):
    dis = _dis_block(dega_ref[...], degb_ref[...])
    g1_ref[...] = xw_ref[...] * dis


def _tc2_body(g1_ref, s1a_ref, s1b_ref, dega_ref, degb_ref, b1_ref, w2_ref,
              g2_ref):
    dis = _dis_block(dega_ref[...], degb_ref[...])
    h1 = dis * (s1a_ref[...] + s1b_ref[...] + g1_ref[...]) + b1_ref[...]
    h1 = jnp.maximum(h1, 0.0)
    g2_ref[...] = jnp.dot(h1, w2_ref[...],
                          preferred_element_type=jnp.float32) * dis


def _tc3_body(g2_ref, s2a_ref, s2b_ref, dega_ref, degb_ref, b2_ref, wc1_ref,
              bc1_ref, a_ref, b_ref):
    dis = _dis_block(dega_ref[...], degb_ref[...])
    h2 = dis * (s2a_ref[...] + s2b_ref[...] + g2_ref[...]) + b2_ref[...]
    wc1 = wc1_ref[...]
    a_ref[...] = jnp.dot(h2, wc1[:H], preferred_element_type=jnp.float32) \
        + bc1_ref[...]
    b_ref[...] = jnp.dot(h2, wc1[H:], preferred_element_type=jnp.float32)


def _row_spec(width):
    return pl.BlockSpec((BN, width), lambda i: (i, 0))


def _full_spec(rows, cols):
    return pl.BlockSpec((rows, cols), lambda i: (0, 0))


_GRID = (NP_ // BN,)

_tc1a = pl.pallas_call(
    _tc1a_body,
    grid=_GRID,
    in_specs=[_row_spec(DIN), _full_spec(DIN, H)],
    out_specs=_row_spec(H),
    out_shape=jax.ShapeDtypeStruct((NP_, H), jnp.float32),
)

_tc1b = pl.pallas_call(
    _tc1b_body,
    grid=_GRID,
    in_specs=[_row_spec(H), _row_spec(16), _row_spec(16)],
    out_specs=_row_spec(H),
    out_shape=jax.ShapeDtypeStruct((NP_, H), jnp.float32),
)

_tc2 = pl.pallas_call(
    _tc2_body,
    grid=_GRID,
    in_specs=[_row_spec(H), _row_spec(H), _row_spec(H), _row_spec(16),
              _row_spec(16), _full_spec(1, H), _full_spec(H, H)],
    out_specs=_row_spec(H),
    out_shape=jax.ShapeDtypeStruct((NP_, H), jnp.float32),
)

_tc3 = pl.pallas_call(
    _tc3_body,
    grid=_GRID,
    in_specs=[_row_spec(H), _row_spec(H), _row_spec(H), _row_spec(16),
              _row_spec(16), _full_spec(1, H), _full_spec(2 * H, H),
              _full_spec(1, H)],
    out_specs=[_row_spec(H), _row_spec(H)],
    out_shape=[jax.ShapeDtypeStruct((NP_, H), jnp.float32),
               jax.ShapeDtypeStruct((NP_, H), jnp.float32)],
)


def kernel(x, edge_index, W1, b1, W2, b2, Wc1, bc1, Wc2, bc2):
    row = edge_index[0]
    col = edge_index[1]
    pad = EP - E
    pad_idx = N + (jnp.arange(pad, dtype=jnp.int32) % (NP_ - N))
    rows3 = jnp.concatenate([row, pad_idx]).reshape(NW, CPT, CK)
    cols3 = jnp.concatenate([col, pad_idx]).reshape(NW, CPT, CK)

    x_p = jnp.zeros((NP_, DIN), jnp.float32).at[:N].set(x)
    ones16 = jnp.ones((CK, 16), jnp.float32)
    zer16 = jnp.zeros((NP_, 16), jnp.float32)
    zer64 = jnp.zeros((NP_, H), jnp.float32)
    wq = Wc2.reshape(H // L, L)
    bc2b = jnp.broadcast_to(bc2.reshape(1), (L,))

    xw = _tc1a(x_p, W1)
    dega, degb = _deg_kernel(cols3, zer16, ones16)

    g1 = _tc1b(xw, dega, degb)
    s1a, s1b = _agg_kernel(g1, rows3, cols3, zer64)
    g2 = _tc2(g1, s1a, s1b, dega, degb, b1.reshape(1, H), W2)
    s2a, s2b = _agg_kernel(g2, rows3, cols3, zer64)
    a_nodes, b_nodes = _tc3(g2, s2a, s2b, dega, degb, b2.reshape(1, H),
                            Wc1, bc1.reshape(1, H))

    out3 = _edge_kernel(a_nodes, b_nodes, rows3, cols3, wq, bc2b)
    return out3.reshape(EP)[:E]


# speedup vs baseline: 1.2667x; 1.0148x over previous
import functools

import jax
import jax.numpy as jnp
import numpy as np
from jax import lax
from jax.experimental import pallas as pl
from jax.experimental.pallas import tpu as pltpu
from jax.experimental.pallas import tpu_sc as plsc

N = 10000
E = 320000
DIN = 128
H = 64

NC = 2
NS = 16
NW = NC * NS
L = 16

CK = 128
CPT = 80
EP = NW * CPT * CK
PAD_IDX = N

NP_ = 10240
RPT = NP_ // NS

NGRP = CK // L

_mesh = plsc.VectorSubcoreMesh(core_axis_name="c", subcore_axis_name="s")
_sc_params = pltpu.CompilerParams(use_tc_tiling_on_sc=False,
                                  needs_layout_passes=False)


def _wids():
    cid = lax.axis_index("c")
    sid = lax.axis_index("s")
    return cid, sid, sid * NC + cid


def _deg_body(cols_hbm, zeros_hbm, ones_hbm, outa_hbm, outb_hbm, idxc_all,
              onesv, acc_sh, sem):
    cid, sid, wid = _wids()
    base = sid * RPT
    pltpu.sync_copy(zeros_hbm.at[pl.ds(base, RPT)], acc_sh.at[pl.ds(base, RPT)])
    pltpu.sync_copy(ones_hbm, onesv)
    pltpu.sync_copy(cols_hbm.at[wid], idxc_all)
    plsc.subcore_barrier()

    @pl.loop(0, CPT)
    def _fire(c):
        pltpu.async_copy(onesv, acc_sh.at[idxc_all.at[c]], sem, add=True)

    @pl.loop(0, CPT)
    def _drain(c):
        pltpu.make_async_copy(onesv, acc_sh.at[idxc_all.at[0]], sem).wait()

    plsc.subcore_barrier()

    @pl.when(cid == 0)
    def _():
        pltpu.sync_copy(acc_sh.at[pl.ds(base, RPT)],
                        outa_hbm.at[pl.ds(base, RPT)])

    @pl.when(cid == 1)
    def _():
        pltpu.sync_copy(acc_sh.at[pl.ds(base, RPT)],
                        outb_hbm.at[pl.ds(base, RPT)])


_deg_kernel = functools.partial(
    pl.kernel,
    out_type=[jax.ShapeDtypeStruct((NP_, 16), jnp.float32),
              jax.ShapeDtypeStruct((NP_, 16), jnp.float32)],
    mesh=_mesh,
    compiler_params=_sc_params,
    scratch_types=[
        pltpu.VMEM((CPT, CK), jnp.int32),
        pltpu.VMEM((CK, 16), jnp.float32),
        pltpu.VMEM_SHARED((NP_, 16), jnp.float32),
        pltpu.SemaphoreType.DMA,
    ],
)(_deg_body)


def _agg_body(g_hbm, rows_hbm, cols_hbm, zeros_hbm, outa_hbm, outb_hbm,
              idxr_all, idxc_all, gbuf, acc_sh, sem_g, sem_s):
    cid, sid, wid = _wids()
    base = sid * RPT
    pltpu.sync_copy(zeros_hbm.at[pl.ds(base, RPT)], acc_sh.at[pl.ds(base, RPT)])
    pltpu.sync_copy(rows_hbm.at[wid], idxr_all)
    pltpu.sync_copy(cols_hbm.at[wid], idxc_all)
    plsc.subcore_barrier()

    def _gather_start(c, b):
        pltpu.async_copy(g_hbm.at[idxr_all.at[c]], gbuf.at[b], sem_g.at[b])

    def _gather_wait(b):
        pltpu.make_async_copy(g_hbm.at[idxr_all.at[0]], gbuf.at[b],
                              sem_g.at[b]).wait()

    def _scatter_start(c, b):
        pltpu.async_copy(gbuf.at[b], acc_sh.at[idxc_all.at[c]], sem_s.at[b],
                         add=True)

    def _scatter_wait(b):
        pltpu.make_async_copy(gbuf.at[b], acc_sh.at[idxc_all.at[0]],
                              sem_s.at[b]).wait()

    _gather_start(0, 0)
    _gather_start(1, 1)

    @pl.loop(0, CPT // 4)
    def _round(r):
        for b in range(4):
            c = r * 4 + b
            b2 = (b + 2) % 4
            @pl.when(c >= 2)
            def _():
                _scatter_wait(b2)

            @pl.when(c + 2 < CPT)
            def _():
                _gather_start(c + 2, b2)

            _gather_wait(b)
            _scatter_start(c, b)

    _scatter_wait((CPT - 2) % 4)
    _scatter_wait((CPT - 1) % 4)
    plsc.subcore_barrier()

    @pl.when(cid == 0)
    def _():
        pltpu.sync_copy(acc_sh.at[pl.ds(base, RPT)],
                        outa_hbm.at[pl.ds(base, RPT)])

    @pl.when(cid == 1)
    def _():
        pltpu.sync_copy(acc_sh.at[pl.ds(base, RPT)],
                        outb_hbm.at[pl.ds(base, RPT)])


_agg_kernel = functools.partial(
    pl.kernel,
    out_type=[jax.ShapeDtypeStruct((NP_, H), jnp.float32),
              jax.ShapeDtypeStruct((NP_, H), jnp.float32)],
    mesh=_mesh,
    compiler_params=_sc_params,
    scratch_types=[
        pltpu.VMEM((CPT, CK), jnp.int32),
        pltpu.VMEM((CPT, CK), jnp.int32),
        pltpu.VMEM((4, CK, H), jnp.float32),
        pltpu.VMEM_SHARED((NP_, H), jnp.float32),
        pltpu.SemaphoreType.DMA((4,)),
        pltpu.SemaphoreType.DMA((4,)),
    ],
)(_agg_body)


def _edge_body(a_hbm, b_hbm, rows_hbm, cols_hbm, wq_hbm, bc2_hbm, out_hbm,
               idxr_all, idxc_all, ebufa, ebufb, wv, bv, outv_all,
               sem_a, sem_b):
    cid, sid, wid = _wids()
    pltpu.sync_copy(wq_hbm, wv)
    pltpu.sync_copy(bc2_hbm, bv)
    pltpu.sync_copy(rows_hbm.at[wid], idxr_all)
    pltpu.sync_copy(cols_hbm.at[wid], idxc_all)

    NB = 4

    def _start(c, b):
        pltpu.async_copy(a_hbm.at[idxr_all.at[c]], ebufa.at[b], sem_a.at[b])
        pltpu.async_copy(b_hbm.at[idxc_all.at[c]], ebufb.at[b], sem_b.at[b])

    def _wait(b):
        pltpu.make_async_copy(a_hbm.at[idxr_all.at[0]], ebufa.at[b],
                              sem_a.at[b]).wait()
        pltpu.make_async_copy(b_hbm.at[idxc_all.at[0]], ebufb.at[b],
                              sem_b.at[b]).wait()

    for b in range(NB):
        _start(b, b)

    L2 = 2 * L
    wq = [wv.at[k][...] for k in range(H // L2)]
    last_lane = lax.iota(jnp.int32, L) == (L - 1)

    @pl.loop(0, CPT // NB)
    def _round(r):
        for b in range(NB):
            c = r * NB + b
            _wait(b)
            cbase = c * CK

            @pl.loop(0, CK, unroll=4)
            def _edge(e):
                acc = None
                for k in range(H // L2):
                    va = ebufa.at[b][e, pl.ds(k * L2, L2)]
                    vb = ebufb.at[b][e, pl.ds(k * L2, L2)]
                    t = jnp.maximum(va + vb, 0.0) * wq[k]
                    acc = t if acc is None else acc + t
                f0, f1 = plsc.unpack(acc, format=plsc.PackFormat.INTERLEAVED,
                                     preferred_element_type=jnp.float32)
                cums = plsc.cumsum(f0 + f1)
                pos = jnp.full((L,), 0, jnp.int32) + (cbase + e)
                plsc.store_scatter(outv_all, [pos], cums, mask=last_lane)

            @pl.when(c + NB < CPT)
            def _():
                _start(c + NB, b)

    bias = bv[...]

    @pl.loop(0, CPT * CK // L)
    def _sig(i):
        s = outv_all[pl.ds(i * L, L)] + bias
        outv_all[pl.ds(i * L, L)] = 1.0 / (1.0 + jnp.exp(-s))

    pltpu.sync_copy(outv_all, out_hbm.at[wid])


_edge_kernel = functools.partial(
    pl.kernel,
    out_type=jax.ShapeDtypeStruct((NW, CPT * CK), jnp.float32),
    mesh=_mesh,
    compiler_params=_sc_params,
    scratch_types=[
        pltpu.VMEM((CPT, CK), jnp.int32),
        pltpu.VMEM((CPT, CK), jnp.int32),
        pltpu.VMEM((4, CK, H), jnp.bfloat16),
        pltpu.VMEM((4, CK, H), jnp.bfloat16),
        pltpu.VMEM((H // (2 * L), 2 * L), jnp.bfloat16),
        pltpu.VMEM((L,), jnp.float32),
        pltpu.VMEM((CPT * CK,), jnp.float32),
        pltpu.SemaphoreType.DMA((4,)),
        pltpu.SemaphoreType.DMA((4,)),
    ],
)(_edge_body)


BN = 2048


def _dis_block(dega, degb):
    deg = dega[:, :1] + degb[:, :1] + 1.0
    return lax.rsqrt(deg)


def _tc1a_body(x_ref, w1_ref, xw_ref):
    xw_ref[...] = jnp.dot(x_ref[...], w1_ref[...],
                          preferred_element_type=jnp.float32)


def _tc1b_body(xw_ref, dega_ref, degb_ref, g1_ref):
    dis = _dis_block(dega_ref[...], degb_ref[...])
    g1_ref[...] = xw_ref[...] * dis


def _tc2_body(g1_ref, s1a_ref, s1b_ref, dega_ref, degb_ref, b1_ref, w2_ref,
              g2_ref):
    dis = _dis_block(dega_ref[...], degb_ref[...])
    h1 = dis * (s1a_ref[...] + s1b_ref[...] + g1_ref[...]) + b1_ref[...]
    h1 = jnp.maximum(h1, 0.0)
    g2_ref[...] = jnp.dot(h1, w2_ref[...],
                          preferred_element_type=jnp.float32) * dis


def _tc3_body(g2_ref, s2a_ref, s2b_ref, dega_ref, degb_ref, b2_ref, wc1_ref,
              bc1_ref, a_ref, b_ref):
    dis = _dis_block(dega_ref[...], degb_ref[...])
    h2 = dis * (s2a_ref[...] + s2b_ref[...] + g2_ref[...]) + b2_ref[...]
    wc1 = wc1_ref[...]
    a_ref[...] = (jnp.dot(h2, wc1[:H], preferred_element_type=jnp.float32)
                  + bc1_ref[...]).astype(jnp.bfloat16)
    b_ref[...] = jnp.dot(h2, wc1[H:],
                         preferred_element_type=jnp.float32).astype(jnp.bfloat16)


def _row_spec(width):
    return pl.BlockSpec((BN, width), lambda i: (i, 0))


def _full_spec(rows, cols):
    return pl.BlockSpec((rows, cols), lambda i: (0, 0))


_GRID = (NP_ // BN,)

_tc1a = pl.pallas_call(
    _tc1a_body,
    grid=_GRID,
    in_specs=[_row_spec(DIN), _full_spec(DIN, H)],
    out_specs=_row_spec(H),
    out_shape=jax.ShapeDtypeStruct((NP_, H), jnp.float32),
)

_tc1b = pl.pallas_call(
    _tc1b_body,
    grid=_GRID,
    in_specs=[_row_spec(H), _row_spec(16), _row_spec(16)],
    out_specs=_row_spec(H),
    out_shape=jax.ShapeDtypeStruct((NP_, H), jnp.float32),
)

_tc2 = pl.pallas_call(
    _tc2_body,
    grid=_GRID,
    in_specs=[_row_spec(H), _row_spec(H), _row_spec(H), _row_spec(16),
              _row_spec(16), _full_spec(1, H), _full_spec(H, H)],
    out_specs=_row_spec(H),
    out_shape=jax.ShapeDtypeStruct((NP_, H), jnp.float32),
)

_tc3 = pl.pallas_call(
    _tc3_body,
    grid=_GRID,
    in_specs=[_row_spec(H), _row_spec(H), _row_spec(H), _row_spec(16),
              _row_spec(16), _full_spec(1, H), _full_spec(2 * H, H),
              _full_spec(1, H)],
    out_specs=[_row_spec(H), _row_spec(H)],
    out_shape=[jax.ShapeDtypeStruct((NP_, H), jnp.bfloat16),
               jax.ShapeDtypeStruct((NP_, H), jnp.bfloat16)],
)


def kernel(x, edge_index, W1, b1, W2, b2, Wc1, bc1, Wc2, bc2):
    row = edge_index[0]
    col = edge_index[1]
    pad = EP - E
    pad_idx = N + (jnp.arange(pad, dtype=jnp.int32) % (NP_ - N))
    rows3 = jnp.concatenate([row, pad_idx]).reshape(NW, CPT, CK)
    cols3 = jnp.concatenate([col, pad_idx]).reshape(NW, CPT, CK)

    x_p = jnp.zeros((NP_, DIN), jnp.float32).at[:N].set(x)
    ones16 = jnp.ones((CK, 16), jnp.float32)
    zer16 = jnp.zeros((NP_, 16), jnp.float32)
    zer64 = jnp.zeros((NP_, H), jnp.float32)
    wq = Wc2.reshape(H // (2 * L), 2 * L).astype(jnp.bfloat16)
    bc2b = jnp.broadcast_to(bc2.reshape(1), (L,))

    xw = _tc1a(x_p, W1)
    dega, degb = _deg_kernel(cols3, zer16, ones16)

    g1 = _tc1b(xw, dega, degb)
    s1a, s1b = _agg_kernel(g1, rows3, cols3, zer64)
    g2 = _tc2(g1, s1a, s1b, dega, degb, b1.reshape(1, H), W2)
    s2a, s2b = _agg_kernel(g2, rows3, cols3, zer64)
    a_nodes, b_nodes = _tc3(g2, s2a, s2b, dega, degb, b2.reshape(1, H),
                            Wc1, bc1.reshape(1, H))

    out3 = _edge_kernel(a_nodes, b_nodes, rows3, cols3, wq, bc2b)
    return out3.reshape(EP)[:E]
